# baseline (speedup 1.0000x reference)
import functools

import numpy as np
import jax
import jax.numpy as jnp
from jax import lax
from jax.experimental import pallas as pl
from jax.experimental.pallas import tpu as pltpu
from jax.experimental.pallas import tpu_sc as plsc

_ALPHA = 0.05
_EPS = 1e-8

_NC = 2
_NS = 16
_NW = _NC * _NS


def _stages(n):
    ln = n.bit_length() - 1
    for lk in range(1, ln + 1):
        for lj in range(lk - 1, -1, -1):
            yield lj, lk


def _roll_masks(n, lj, lk):
    i = lax.broadcasted_iota(jnp.int32, (n, 1), 0)
    bitj = ((i >> lj) & 1) == 1
    take_min = (((i >> lj) ^ (i >> lk)) & 1) == 0
    return bitj, take_min


def _asc_mask(m, lj, lk):
    bm = lax.broadcasted_iota(jnp.int32, (m, 1, 1), 0)
    return ((bm >> (lk - lj - 1)) & 1) == 0


def _stage_val(x, lj, lk):
    n, c = x.shape
    j = 1 << lj
    if lj >= 3:
        m = n // (2 * j)
        x3 = x.reshape(m, 2, j, c)
        ah, bh = x3[:, 0], x3[:, 1]
        mn = jnp.minimum(ah, bh)
        mx = jnp.maximum(ah, bh)
        asc = _asc_mask(m, lj, lk)
        na = jnp.where(asc, mn, mx)
        nb = jnp.where(asc, mx, mn)
        return jnp.stack([na, nb], axis=1).reshape(n, c)
    bitj, tm = _roll_masks(n, lj, lk)
    p = jnp.where(bitj, pltpu.roll(x, j, 0), pltpu.roll(x, n - j, 0))
    return jnp.where((x < p) == tm, x, p)


def _sortable(f):
    b = lax.bitcast_convert_type(f, jnp.int32)
    return b ^ ((b >> 31) & jnp.int32(0x7FFFFFFF))


def _merge_rec(v, asc, ell):
    j = ell >> 1
    a, b = v[:, :j], v[:, j:]
    mn = jnp.minimum(a, b)
    mx = jnp.maximum(a, b)
    na = jnp.where(asc, mn, mx)
    nb = jnp.where(asc, mx, mn)
    if j <= 16:
        return [na, nb]
    return _merge_rec(na, asc, j) + _merge_rec(nb, asc, j)


def _sort_keys(x):
    n, c = x.shape
    ln = n.bit_length() - 1
    for lk in range(1, ln + 1):
        if lk <= 3:
            for lj in range(lk - 1, -1, -1):
                x = _stage_val(x, lj, lk)
        else:
            nb = n >> lk
            v = x.reshape(nb, 1 << lk, c)
            bm = lax.broadcasted_iota(jnp.int32, (nb, 1, 1), 0)
            pieces = _merge_rec(v, (bm & 1) == 0, 1 << lk)
            x = jnp.stack(pieces, axis=1).reshape(n, c)
            start = 3 if lk >= 5 else 2
            for lj in range(start, -1, -1):
                x = _stage_val(x, lj, lk)
    return x


def _sort_old_body(old_ref, k1_ref):
    k1_ref[...] = _sort_keys(_sortable(old_ref[...]))


def _sort_aligned_body(aligned_ref, k2_ref):
    n = aligned_ref.shape[0]
    rb = n.bit_length() - 1
    rows = lax.broadcasted_iota(jnp.int32, (n, 1), 0)
    k2 = (_sortable(aligned_ref[...]) & jnp.int32(~((1 << rb) - 1))) | rows
    k2_ref[...] = _sort_keys(k2)


def _scatter_combine_body(k1_ref, k2_ref, aligned_ref, new_ref, out_ref):
    a = _ALPHA
    n = k1_ref.shape[0]
    rb = n.bit_length() - 1
    pb = 32 - rb
    idx_tgt = k2_ref[...] & jnp.int32((1 << rb) - 1)
    pay = (k1_ref[...] >> rb) & jnp.int32((1 << pb) - 1)
    k3 = _sort_keys(((idx_tgt - jnp.int32(n // 2)) << pb) | pay)
    sb = (k3 & jnp.int32((1 << pb) - 1)) << rb
    s = lax.bitcast_convert_type(sb ^ ((sb >> 31) & jnp.int32(0x7FFFFFFF)),
                                 jnp.float32)
    out_ref[...] = ((1.0 - a) * new_ref[...]
                    + (a - a * a) * aligned_ref[...]
                    + (a * a) * s)


def _norm_body(x_ref, y_ref):
    x = x_ref[...]
    nrm = jnp.sqrt(jnp.sum(x * x, axis=1, keepdims=True))
    y_ref[...] = x / jnp.maximum(nrm, _EPS)


def _argmax_body(old_ref, newn_ref, idx_ref):
    sim = lax.dot_general(old_ref[...], newn_ref[...],
                          (((1,), (1,)), ((), ())),
                          preferred_element_type=jnp.float32)
    m = jnp.max(sim, axis=1, keepdims=True)
    col = lax.broadcasted_iota(jnp.int32, sim.shape, 1)
    cand = jnp.where(sim == m, col, sim.shape[1])
    idx_ref[...] = jnp.min(cand, axis=1, keepdims=True)


def _normalize(x):
    n, d = x.shape
    blk = min(n, 512)
    return pl.pallas_call(
        _norm_body,
        grid=(n // blk,),
        in_specs=[pl.BlockSpec((blk, d), lambda i: (i, 0))],
        out_specs=pl.BlockSpec((blk, d), lambda i: (i, 0)),
        out_shape=jax.ShapeDtypeStruct((n, d), jnp.float32),
    )(x)


def _top1(hidden_old, new_n):
    n, d = hidden_old.shape
    m = new_n.shape[0]
    blk = 256
    idx = pl.pallas_call(
        _argmax_body,
        grid=(n // blk,),
        in_specs=[pl.BlockSpec((blk, d), lambda i: (i, 0)),
                  pl.BlockSpec((m, d), lambda i: (0, 0))],
        out_specs=pl.BlockSpec((blk, 1), lambda i: (i, 0)),
        out_shape=jax.ShapeDtypeStruct((n, 1), jnp.int32),
    )(hidden_old, new_n)
    return idx.reshape(n)


def _colspec(n, blk):
    return pl.BlockSpec((n, blk), lambda i: (0, i))


def _rowspec(n, blk):
    return pl.BlockSpec((blk, n), lambda i: (i, 0))


def _ot_update(hidden_old, aligned, hidden_new, blk=128):
    n, d = aligned.shape
    k1 = pl.pallas_call(
        _sort_old_body,
        grid=(d // blk,),
        in_specs=[_colspec(n, blk)],
        out_specs=_colspec(n, blk),
        out_shape=jax.ShapeDtypeStruct((n, d), jnp.int32),
    )(hidden_old)
    k2 = pl.pallas_call(
        _sort_aligned_body,
        grid=(d // blk,),
        in_specs=[_colspec(n, blk)],
        out_specs=_colspec(n, blk),
        out_shape=jax.ShapeDtypeStruct((n, d), jnp.int32),
    )(aligned)
    return pl.pallas_call(
        _scatter_combine_body,
        grid=(d // blk,),
        in_specs=[_colspec(n, blk)] * 4,
        out_specs=_colspec(n, blk),
        out_shape=jax.ShapeDtypeStruct((n, d), jnp.float32),
    )(k1, k2, aligned, hidden_new)


def _gather_rows(table, idx):
    n, d = table.shape
    b = idx.shape[0]
    bpw = b // _NW
    mesh = plsc.VectorSubcoreMesh(core_axis_name="c", subcore_axis_name="s")

    @functools.partial(
        pl.kernel, mesh=mesh,
        out_type=jax.ShapeDtypeStruct((b, d), jnp.float32),
        scratch_types=[
            pltpu.VMEM((bpw,), jnp.int32),
            pltpu.VMEM((bpw, d), jnp.float32),
            pltpu.SemaphoreType.DMA,
        ],
    )
    def k(table_hbm, idx_hbm, out_hbm, idx_v, rows_v, sem):
        wid = lax.axis_index("s") * _NC + lax.axis_index("c")
        base = wid * bpw
        pltpu.sync_copy(idx_hbm.at[pl.ds(base, bpw)], idx_v)
        pltpu.async_copy(table_hbm.at[idx_v], rows_v, sem).wait()
        pltpu.sync_copy(rows_v, out_hbm.at[pl.ds(base, bpw)])

    return k(table, idx)


def kernel(hidden_old, hidden_new):
    new_n = _normalize(hidden_new)
    idx = _top1(hidden_old, new_n)
    aligned = _gather_rows(hidden_new, idx)
    return _ot_update(hidden_old, aligned, hidden_new)


# speedup vs baseline: 1.5205x; 1.5205x over previous
import functools

import numpy as np
import jax
import jax.numpy as jnp
from jax import lax
from jax.experimental import pallas as pl
from jax.experimental.pallas import tpu as pltpu
from jax.experimental.pallas import tpu_sc as plsc

_ALPHA = 0.05
_EPS = 1e-8

_NC = 2
_NS = 16
_NW = _NC * _NS


def _stages(n):
    ln = n.bit_length() - 1
    for lk in range(1, ln + 1):
        for lj in range(lk - 1, -1, -1):
            yield lj, lk


def _tile_stage(v, lj, kb):
    j = 1 << lj
    r = lax.broadcasted_iota(jnp.int32, (1, 8, 1), 1)
    bj = (r >> lj) & 1
    tm = bj == kb
    p = jnp.where(bj == 1, pltpu.roll(v, j, 1), pltpu.roll(v, 8 - j, 1))
    return jnp.where((v < p) == tm, v, p)


def _sortable(f):
    b = lax.bitcast_convert_type(f, jnp.int32)
    return b ^ ((b >> 31) & jnp.int32(0x7FFFFFFF))


def _merge_rec(v, asc, ell):
    j = ell >> 1
    a, b = v[:, :j], v[:, j:]
    mn = jnp.minimum(a, b)
    mx = jnp.maximum(a, b)
    na = jnp.where(asc, mn, mx)
    nb = jnp.where(asc, mx, mn)
    if j == 8:
        return [na, nb]
    return _merge_rec(na, asc, j) + _merge_rec(nb, asc, j)


def _sort_keys(x):
    n, c = x.shape
    ln = n.bit_length() - 1
    nb8 = n // 8
    r = lax.broadcasted_iota(jnp.int32, (1, 8, 1), 1)
    t = lax.broadcasted_iota(jnp.int32, (nb8, 1, 1), 0)
    v = x.reshape(nb8, 8, c)
    for lk in (1, 2, 3):
        kb = ((r >> lk) & 1) if lk < 3 else (t & 1)
        for lj in range(lk - 1, -1, -1):
            v = _tile_stage(v, lj, kb)
    x = v.reshape(n, c)
    for lk in range(4, ln + 1):
        nb = n >> lk
        z = x.reshape(nb, 1 << lk, c)
        bm = lax.broadcasted_iota(jnp.int32, (nb, 1, 1), 0)
        kb = bm & 1
        pieces = _merge_rec(z, kb == 0, 1 << lk)
        done = []
        for p in pieces:
            for lj in (2, 1, 0):
                p = _tile_stage(p, lj, kb)
            done.append(p)
        x = jnp.stack(done, axis=1).reshape(n, c)
    return x


def _sort_old_body(old_ref, k1_ref):
    k1_ref[...] = _sort_keys(_sortable(old_ref[...]))


def _sort_aligned_body(aligned_ref, k2_ref):
    n = aligned_ref.shape[0]
    rb = n.bit_length() - 1
    rows = lax.broadcasted_iota(jnp.int32, (n, 1), 0)
    k2 = (_sortable(aligned_ref[...]) & jnp.int32(~((1 << rb) - 1))) | rows
    k2_ref[...] = _sort_keys(k2)


def _scatter_combine_body(k1_ref, k2_ref, aligned_ref, new_ref, out_ref):
    a = _ALPHA
    n = k1_ref.shape[0]
    rb = n.bit_length() - 1
    pb = 32 - rb
    idx_tgt = k2_ref[...] & jnp.int32((1 << rb) - 1)
    pay = (k1_ref[...] >> rb) & jnp.int32((1 << pb) - 1)
    k3 = _sort_keys(((idx_tgt - jnp.int32(n // 2)) << pb) | pay)
    sb = (k3 & jnp.int32((1 << pb) - 1)) << rb
    s = lax.bitcast_convert_type(sb ^ ((sb >> 31) & jnp.int32(0x7FFFFFFF)),
                                 jnp.float32)
    out_ref[...] = ((1.0 - a) * new_ref[...]
                    + (a - a * a) * aligned_ref[...]
                    + (a * a) * s)


def _norm_body(x_ref, y_ref):
    x = x_ref[...]
    nrm = jnp.sqrt(jnp.sum(x * x, axis=1, keepdims=True))
    y_ref[...] = x / jnp.maximum(nrm, _EPS)


def _argmax_body(old_ref, newn_ref, idx_ref):
    sim = lax.dot_general(old_ref[...], newn_ref[...],
                          (((1,), (1,)), ((), ())),
                          preferred_element_type=jnp.float32)
    m = jnp.max(sim, axis=1, keepdims=True)
    col = lax.broadcasted_iota(jnp.int32, sim.shape, 1)
    cand = jnp.where(sim == m, col, sim.shape[1])
    idx_ref[...] = jnp.min(cand, axis=1, keepdims=True)


def _normalize(x):
    n, d = x.shape
    blk = min(n, 512)
    return pl.pallas_call(
        _norm_body,
        grid=(n // blk,),
        in_specs=[pl.BlockSpec((blk, d), lambda i: (i, 0))],
        out_specs=pl.BlockSpec((blk, d), lambda i: (i, 0)),
        out_shape=jax.ShapeDtypeStruct((n, d), jnp.float32),
    )(x)


def _top1(hidden_old, new_n):
    n, d = hidden_old.shape
    m = new_n.shape[0]
    blk = 256
    idx = pl.pallas_call(
        _argmax_body,
        grid=(n // blk,),
        in_specs=[pl.BlockSpec((blk, d), lambda i: (i, 0)),
                  pl.BlockSpec((m, d), lambda i: (0, 0))],
        out_specs=pl.BlockSpec((blk, 1), lambda i: (i, 0)),
        out_shape=jax.ShapeDtypeStruct((n, 1), jnp.int32),
    )(hidden_old, new_n)
    return idx.reshape(n)


def _colspec(n, blk):
    return pl.BlockSpec((n, blk), lambda i: (0, i))


def _rowspec(n, blk):
    return pl.BlockSpec((blk, n), lambda i: (i, 0))


def _ot_update(hidden_old, aligned, hidden_new, blk=128):
    n, d = aligned.shape
    k1 = pl.pallas_call(
        _sort_old_body,
        grid=(d // blk,),
        in_specs=[_colspec(n, blk)],
        out_specs=_colspec(n, blk),
        out_shape=jax.ShapeDtypeStruct((n, d), jnp.int32),
    )(hidden_old)
    k2 = pl.pallas_call(
        _sort_aligned_body,
        grid=(d // blk,),
        in_specs=[_colspec(n, blk)],
        out_specs=_colspec(n, blk),
        out_shape=jax.ShapeDtypeStruct((n, d), jnp.int32),
    )(aligned)
    return pl.pallas_call(
        _scatter_combine_body,
        grid=(d // blk,),
        in_specs=[_colspec(n, blk)] * 4,
        out_specs=_colspec(n, blk),
        out_shape=jax.ShapeDtypeStruct((n, d), jnp.float32),
    )(k1, k2, aligned, hidden_new)


def _gather_rows(table, idx):
    n, d = table.shape
    b = idx.shape[0]
    bpw = b // _NW
    mesh = plsc.VectorSubcoreMesh(core_axis_name="c", subcore_axis_name="s")

    @functools.partial(
        pl.kernel, mesh=mesh,
        out_type=jax.ShapeDtypeStruct((b, d), jnp.float32),
        scratch_types=[
            pltpu.VMEM((bpw,), jnp.int32),
            pltpu.VMEM((bpw, d), jnp.float32),
            pltpu.SemaphoreType.DMA,
        ],
    )
    def k(table_hbm, idx_hbm, out_hbm, idx_v, rows_v, sem):
        wid = lax.axis_index("s") * _NC + lax.axis_index("c")
        base = wid * bpw
        pltpu.sync_copy(idx_hbm.at[pl.ds(base, bpw)], idx_v)
        pltpu.async_copy(table_hbm.at[idx_v], rows_v, sem).wait()
        pltpu.sync_copy(rows_v, out_hbm.at[pl.ds(base, bpw)])

    return k(table, idx)


def kernel(hidden_old, hidden_new):
    new_n = _normalize(hidden_new)
    idx = _top1(hidden_old, new_n)
    aligned = _gather_rows(hidden_new, idx)
    return _ot_update(hidden_old, aligned, hidden_new)


# speedup vs baseline: 1.6403x; 1.0788x over previous
import functools

import numpy as np
import jax
import jax.numpy as jnp
from jax import lax
from jax.experimental import pallas as pl
from jax.experimental.pallas import tpu as pltpu
from jax.experimental.pallas import tpu_sc as plsc

_ALPHA = 0.05
_EPS = 1e-8

_NC = 2
_NS = 16
_NW = _NC * _NS


def _stages(n):
    ln = n.bit_length() - 1
    for lk in range(1, ln + 1):
        for lj in range(lk - 1, -1, -1):
            yield lj, lk


def _tile_stage(v, lj, kb):
    j = 1 << lj
    r = lax.broadcasted_iota(jnp.int32, (1, 8, 1), 1)
    bj = (r >> lj) & 1
    tm = bj == kb
    p = jnp.where(bj == 1, pltpu.roll(v, j, 1), pltpu.roll(v, 8 - j, 1))
    return jnp.where((v < p) == tm, v, p)


def _sortable(f):
    b = lax.bitcast_convert_type(f, jnp.int32)
    return b ^ ((b >> 31) & jnp.int32(0x7FFFFFFF))


def _merge_rec(v, asc, ell):
    j = ell >> 1
    a, b = v[:, :j], v[:, j:]
    mn = jnp.minimum(a, b)
    mx = jnp.maximum(a, b)
    na = jnp.where(asc, mn, mx)
    nb = jnp.where(asc, mx, mn)
    if j == 8:
        return [na, nb]
    return _merge_rec(na, asc, j) + _merge_rec(nb, asc, j)


def _sort_keys(x):
    n, c = x.shape
    ln = n.bit_length() - 1
    nb8 = n // 8
    r = lax.broadcasted_iota(jnp.int32, (1, 8, 1), 1)
    t = lax.broadcasted_iota(jnp.int32, (nb8, 1, 1), 0)
    v = x.reshape(nb8, 8, c)
    for lk in (1, 2, 3):
        kb = ((r >> lk) & 1) if lk < 3 else (t & 1)
        for lj in range(lk - 1, -1, -1):
            v = _tile_stage(v, lj, kb)
    x = v.reshape(n, c)
    for lk in range(4, ln + 1):
        nb = n >> lk
        z = x.reshape(nb, 1 << lk, c)
        bm = lax.broadcasted_iota(jnp.int32, (nb, 1, 1), 0)
        kb = bm & 1
        pieces = _merge_rec(z, kb == 0, 1 << lk)
        done = []
        for p in pieces:
            for lj in (2, 1, 0):
                p = _tile_stage(p, lj, kb)
            done.append(p)
        x = jnp.stack(done, axis=1).reshape(n, c)
    return x


def _sort_old_body(old_ref, k1_ref):
    k1_ref[...] = _sort_keys(_sortable(old_ref[...]))


def _sort_aligned_body(aligned_ref, k2_ref):
    n = aligned_ref.shape[0]
    rb = n.bit_length() - 1
    rows = lax.broadcasted_iota(jnp.int32, (n, 1), 0)
    k2 = (_sortable(aligned_ref[...]) & jnp.int32(~((1 << rb) - 1))) | rows
    k2_ref[...] = _sort_keys(k2)


def _scatter_combine_body(k1_ref, k2_ref, aligned_ref, new_ref, out_ref):
    a = _ALPHA
    n = k1_ref.shape[0]
    rb = n.bit_length() - 1
    pb = 32 - rb
    idx_tgt = k2_ref[...] & jnp.int32((1 << rb) - 1)
    pay = (k1_ref[...] >> rb) & jnp.int32((1 << pb) - 1)
    k3 = _sort_keys(((idx_tgt - jnp.int32(n // 2)) << pb) | pay)
    sb = (k3 & jnp.int32((1 << pb) - 1)) << rb
    s = lax.bitcast_convert_type(sb ^ ((sb >> 31) & jnp.int32(0x7FFFFFFF)),
                                 jnp.float32)
    out_ref[...] = ((1.0 - a) * new_ref[...]
                    + (a - a * a) * aligned_ref[...]
                    + (a * a) * s)


def _norm_body(x_ref, y_ref):
    x = x_ref[...]
    nrm = jnp.sqrt(jnp.sum(x * x, axis=1, keepdims=True))
    y_ref[...] = x / jnp.maximum(nrm, _EPS)


def _argmax_body(old_ref, newn_ref, idx_ref):
    sim = lax.dot_general(old_ref[...], newn_ref[...],
                          (((1,), (1,)), ((), ())),
                          preferred_element_type=jnp.float32)
    m = jnp.max(sim, axis=1, keepdims=True)
    col = lax.broadcasted_iota(jnp.int32, sim.shape, 1)
    cand = jnp.where(sim == m, col, sim.shape[1])
    idx_ref[...] = jnp.min(cand, axis=1, keepdims=True)


def _normalize(x):
    n, d = x.shape
    blk = min(n, 512)
    return pl.pallas_call(
        _norm_body,
        grid=(n // blk,),
        in_specs=[pl.BlockSpec((blk, d), lambda i: (i, 0))],
        out_specs=pl.BlockSpec((blk, d), lambda i: (i, 0)),
        out_shape=jax.ShapeDtypeStruct((n, d), jnp.float32),
    )(x)


def _top1(hidden_old, new_n):
    n, d = hidden_old.shape
    m = new_n.shape[0]
    blk = 256
    idx = pl.pallas_call(
        _argmax_body,
        grid=(n // blk,),
        in_specs=[pl.BlockSpec((blk, d), lambda i: (i, 0)),
                  pl.BlockSpec((m, d), lambda i: (0, 0))],
        out_specs=pl.BlockSpec((blk, 1), lambda i: (i, 0)),
        out_shape=jax.ShapeDtypeStruct((n, 1), jnp.int32),
    )(hidden_old, new_n)
    return idx.reshape(n)


def _colspec(n, blk):
    return pl.BlockSpec((n, blk), lambda i: (0, i))


def _rowspec(n, blk):
    return pl.BlockSpec((blk, n), lambda i: (i, 0))


def _ot_update(hidden_old, aligned, hidden_new, blk=256):
    n, d = aligned.shape
    k1 = pl.pallas_call(
        _sort_old_body,
        grid=(d // blk,),
        in_specs=[_colspec(n, blk)],
        out_specs=_colspec(n, blk),
        out_shape=jax.ShapeDtypeStruct((n, d), jnp.int32),
    )(hidden_old)
    k2 = pl.pallas_call(
        _sort_aligned_body,
        grid=(d // blk,),
        in_specs=[_colspec(n, blk)],
        out_specs=_colspec(n, blk),
        out_shape=jax.ShapeDtypeStruct((n, d), jnp.int32),
    )(aligned)
    return pl.pallas_call(
        _scatter_combine_body,
        grid=(d // blk,),
        in_specs=[_colspec(n, blk)] * 4,
        out_specs=_colspec(n, blk),
        out_shape=jax.ShapeDtypeStruct((n, d), jnp.float32),
    )(k1, k2, aligned, hidden_new)


def _gather_rows(table, idx):
    n, d = table.shape
    b = idx.shape[0]
    bpw = b // _NW
    mesh = plsc.VectorSubcoreMesh(core_axis_name="c", subcore_axis_name="s")

    @functools.partial(
        pl.kernel, mesh=mesh,
        out_type=jax.ShapeDtypeStruct((b, d), jnp.float32),
        scratch_types=[
            pltpu.VMEM((bpw,), jnp.int32),
            pltpu.VMEM((bpw, d), jnp.float32),
            pltpu.SemaphoreType.DMA,
        ],
    )
    def k(table_hbm, idx_hbm, out_hbm, idx_v, rows_v, sem):
        wid = lax.axis_index("s") * _NC + lax.axis_index("c")
        base = wid * bpw
        pltpu.sync_copy(idx_hbm.at[pl.ds(base, bpw)], idx_v)
        pltpu.async_copy(table_hbm.at[idx_v], rows_v, sem).wait()
        pltpu.sync_copy(rows_v, out_hbm.at[pl.ds(base, bpw)])

    return k(table, idx)


def kernel(hidden_old, hidden_new):
    new_n = _normalize(hidden_new)
    idx = _top1(hidden_old, new_n)
    aligned = _gather_rows(hidden_new, idx)
    return _ot_update(hidden_old, aligned, hidden_new)


# speedup vs baseline: 2.0262x; 1.2352x over previous
import functools

import numpy as np
import jax
import jax.numpy as jnp
from jax import lax
from jax.experimental import pallas as pl
from jax.experimental.pallas import tpu as pltpu
from jax.experimental.pallas import tpu_sc as plsc

_ALPHA = 0.05
_EPS = 1e-8

_NC = 2
_NS = 16
_NW = _NC * _NS


def _stages(n):
    ln = n.bit_length() - 1
    for lk in range(1, ln + 1):
        for lj in range(lk - 1, -1, -1):
            yield lj, lk


def _tile_stage(v, lj, kb):
    j = 1 << lj
    r = lax.broadcasted_iota(jnp.int32, (1, 8, 1), 1)
    bj = (r >> lj) & 1
    tm = bj == kb
    p = jnp.where(bj == 1, pltpu.roll(v, j, 1), pltpu.roll(v, 8 - j, 1))
    return jnp.where((v < p) == tm, v, p)


def _sortable(f):
    b = lax.bitcast_convert_type(f, jnp.int32)
    return b ^ ((b >> 31) & jnp.int32(0x7FFFFFFF))


def _merge_rec(v, asc, ell):
    j = ell >> 1
    a, b = v[:, :j], v[:, j:]
    mn = jnp.minimum(a, b)
    mx = jnp.maximum(a, b)
    na = jnp.where(asc, mn, mx)
    nb = jnp.where(asc, mx, mn)
    if j == 8:
        return [na, nb]
    return _merge_rec(na, asc, j) + _merge_rec(nb, asc, j)


def _sort_keys(x):
    n, c = x.shape
    ln = n.bit_length() - 1
    nb8 = n // 8
    r = lax.broadcasted_iota(jnp.int32, (1, 8, 1), 1)
    t = lax.broadcasted_iota(jnp.int32, (nb8, 1, 1), 0)
    v = x.reshape(nb8, 8, c)
    for lk in (1, 2, 3):
        kb = ((r >> lk) & 1) if lk < 3 else (t & 1)
        for lj in range(lk - 1, -1, -1):
            v = _tile_stage(v, lj, kb)
    x = v.reshape(n, c)
    for lk in range(4, ln + 1):
        nb = n >> lk
        z = x.reshape(nb, 1 << lk, c)
        bm = lax.broadcasted_iota(jnp.int32, (nb, 1, 1), 0)
        kb = bm & 1
        pieces = _merge_rec(z, kb == 0, 1 << lk)
        done = []
        for p in pieces:
            for lj in (2, 1, 0):
                p = _tile_stage(p, lj, kb)
            done.append(p)
        x = jnp.stack(done, axis=1).reshape(n, c)
    return x


def _sort_old_body(old_ref, k1_ref):
    k1_ref[...] = _sort_keys(_sortable(old_ref[...]))


def _sort_aligned_body(aligned_ref, k2_ref):
    n = aligned_ref.shape[0]
    rb = n.bit_length() - 1
    rows = lax.broadcasted_iota(jnp.int32, (n, 1), 0)
    k2 = (_sortable(aligned_ref[...]) & jnp.int32(~((1 << rb) - 1))) | rows
    k2_ref[...] = _sort_keys(k2)


def _scatter_combine_body(k1_ref, k2_ref, aligned_ref, new_ref, out_ref):
    a = _ALPHA
    n = k1_ref.shape[0]
    rb = n.bit_length() - 1
    pb = 32 - rb
    idx_tgt = k2_ref[...] & jnp.int32((1 << rb) - 1)
    pay = (k1_ref[...] >> rb) & jnp.int32((1 << pb) - 1)
    k3 = _sort_keys(((idx_tgt - jnp.int32(n // 2)) << pb) | pay)
    sb = (k3 & jnp.int32((1 << pb) - 1)) << rb
    s = lax.bitcast_convert_type(sb ^ ((sb >> 31) & jnp.int32(0x7FFFFFFF)),
                                 jnp.float32)
    out_ref[...] = ((1.0 - a) * new_ref[...]
                    + (a - a * a) * aligned_ref[...]
                    + (a * a) * s)


def _norm_body(x_ref, y_ref):
    x = x_ref[...]
    nrm = jnp.sqrt(jnp.sum(x * x, axis=1, keepdims=True))
    y_ref[...] = x / jnp.maximum(nrm, _EPS)


def _argmax_body(old_ref, newn_ref, idx_ref):
    sim = lax.dot_general(old_ref[...], newn_ref[...],
                          (((1,), (1,)), ((), ())),
                          preferred_element_type=jnp.float32)
    m = jnp.max(sim, axis=1, keepdims=True)
    col = lax.broadcasted_iota(jnp.int32, sim.shape, 1)
    cand = jnp.where(sim == m, col, sim.shape[1])
    idx_ref[...] = jnp.min(cand, axis=1, keepdims=True)


def _normalize(x):
    n, d = x.shape
    blk = min(n, 512)
    return pl.pallas_call(
        _norm_body,
        grid=(n // blk,),
        in_specs=[pl.BlockSpec((blk, d), lambda i: (i, 0))],
        out_specs=pl.BlockSpec((blk, d), lambda i: (i, 0)),
        out_shape=jax.ShapeDtypeStruct((n, d), jnp.float32),
    )(x)


def _top1(hidden_old, new_n):
    n, d = hidden_old.shape
    m = new_n.shape[0]
    blk = 256
    idx = pl.pallas_call(
        _argmax_body,
        grid=(n // blk,),
        in_specs=[pl.BlockSpec((blk, d), lambda i: (i, 0)),
                  pl.BlockSpec((m, d), lambda i: (0, 0))],
        out_specs=pl.BlockSpec((blk, 1), lambda i: (i, 0)),
        out_shape=jax.ShapeDtypeStruct((n, 1), jnp.int32),
    )(hidden_old, new_n)
    return idx.reshape(n)


def _colspec(n, blk):
    return pl.BlockSpec((n, blk), lambda i: (0, i))


def _rowspec(n, blk):
    return pl.BlockSpec((blk, n), lambda i: (i, 0))


def _ot_update(hidden_old, aligned, hidden_new, blk=512):
    n, d = aligned.shape
    k1 = pl.pallas_call(
        _sort_old_body,
        grid=(d // blk,),
        in_specs=[_colspec(n, blk)],
        out_specs=_colspec(n, blk),
        out_shape=jax.ShapeDtypeStruct((n, d), jnp.int32),
    )(hidden_old)
    k2 = pl.pallas_call(
        _sort_aligned_body,
        grid=(d // blk,),
        in_specs=[_colspec(n, blk)],
        out_specs=_colspec(n, blk),
        out_shape=jax.ShapeDtypeStruct((n, d), jnp.int32),
    )(aligned)
    return pl.pallas_call(
        _scatter_combine_body,
        grid=(d // blk,),
        in_specs=[_colspec(n, blk)] * 4,
        out_specs=_colspec(n, blk),
        out_shape=jax.ShapeDtypeStruct((n, d), jnp.float32),
    )(k1, k2, aligned, hidden_new)


def _gather_rows(table, idx):
    n, d = table.shape
    b = idx.shape[0]
    bpw = b // _NW
    mesh = plsc.VectorSubcoreMesh(core_axis_name="c", subcore_axis_name="s")

    @functools.partial(
        pl.kernel, mesh=mesh,
        out_type=jax.ShapeDtypeStruct((b, d), jnp.float32),
        scratch_types=[
            pltpu.VMEM((bpw,), jnp.int32),
            pltpu.VMEM((bpw, d), jnp.float32),
            pltpu.SemaphoreType.DMA,
        ],
    )
    def k(table_hbm, idx_hbm, out_hbm, idx_v, rows_v, sem):
        wid = lax.axis_index("s") * _NC + lax.axis_index("c")
        base = wid * bpw
        pltpu.sync_copy(idx_hbm.at[pl.ds(base, bpw)], idx_v)
        pltpu.async_copy(table_hbm.at[idx_v], rows_v, sem).wait()
        pltpu.sync_copy(rows_v, out_hbm.at[pl.ds(base, bpw)])

    return k(table, idx)


def kernel(hidden_old, hidden_new):
    new_n = _normalize(hidden_new)
    idx = _top1(hidden_old, new_n)
    aligned = _gather_rows(hidden_new, idx)
    return _ot_update(hidden_old, aligned, hidden_new)


# speedup vs baseline: 2.0370x; 1.0053x over previous
import functools

import numpy as np
import jax
import jax.numpy as jnp
from jax import lax
from jax.experimental import pallas as pl
from jax.experimental.pallas import tpu as pltpu
from jax.experimental.pallas import tpu_sc as plsc

_ALPHA = 0.05
_EPS = 1e-8

_NC = 2
_NS = 16
_NW = _NC * _NS


def _stages(n):
    ln = n.bit_length() - 1
    for lk in range(1, ln + 1):
        for lj in range(lk - 1, -1, -1):
            yield lj, lk


def _tile_stage(v, lj, kb):
    j = 1 << lj
    r = lax.broadcasted_iota(jnp.int32, (1, 8, 1), 1)
    bj = (r >> lj) & 1
    tm = bj == kb
    p = jnp.where(bj == 1, pltpu.roll(v, j, 1), pltpu.roll(v, 8 - j, 1))
    return jnp.where((v < p) == tm, v, p)


def _sortable(f):
    b = lax.bitcast_convert_type(f, jnp.int32)
    return b ^ ((b >> 31) & jnp.int32(0x7FFFFFFF))


def _merge_rec(v, asc, ell):
    j = ell >> 1
    a, b = v[:, :j], v[:, j:]
    mn = jnp.minimum(a, b)
    mx = jnp.maximum(a, b)
    na = jnp.where(asc, mn, mx)
    nb = jnp.where(asc, mx, mn)
    if j == 8:
        return [na, nb]
    return _merge_rec(na, asc, j) + _merge_rec(nb, asc, j)


def _sort_keys(x):
    n, c = x.shape
    ln = n.bit_length() - 1
    nb8 = n // 8
    r = lax.broadcasted_iota(jnp.int32, (1, 8, 1), 1)
    t = lax.broadcasted_iota(jnp.int32, (nb8, 1, 1), 0)
    v = x.reshape(nb8, 8, c)
    for lk in (1, 2, 3):
        kb = ((r >> lk) & 1) if lk < 3 else (t & 1)
        for lj in range(lk - 1, -1, -1):
            v = _tile_stage(v, lj, kb)
    x = v.reshape(n, c)
    for lk in range(4, ln + 1):
        nb = n >> lk
        z = x.reshape(nb, 1 << lk, c)
        bm = lax.broadcasted_iota(jnp.int32, (nb, 1, 1), 0)
        kb = bm & 1
        pieces = _merge_rec(z, kb == 0, 1 << lk)
        done = []
        for p in pieces:
            for lj in (2, 1, 0):
                p = _tile_stage(p, lj, kb)
            done.append(p)
        x = jnp.stack(done, axis=1).reshape(n, c)
    return x


def _sort_old_body(old_ref, k1_ref):
    k1_ref[...] = _sort_keys(_sortable(old_ref[...]))


def _sort_aligned_body(aligned_ref, k2_ref):
    n = aligned_ref.shape[0]
    rb = n.bit_length() - 1
    rows = lax.broadcasted_iota(jnp.int32, (n, 1), 0)
    k2 = (_sortable(aligned_ref[...]) & jnp.int32(~((1 << rb) - 1))) | rows
    k2_ref[...] = _sort_keys(k2)


def _scatter_combine_body(k1_ref, k2_ref, aligned_ref, new_ref, out_ref):
    a = _ALPHA
    n = k1_ref.shape[0]
    rb = n.bit_length() - 1
    pb = 32 - rb
    idx_tgt = k2_ref[...] & jnp.int32((1 << rb) - 1)
    pay = (k1_ref[...] >> rb) & jnp.int32((1 << pb) - 1)
    k3 = _sort_keys(((idx_tgt - jnp.int32(n // 2)) << pb) | pay)
    sb = (k3 & jnp.int32((1 << pb) - 1)) << rb
    s = lax.bitcast_convert_type(sb ^ ((sb >> 31) & jnp.int32(0x7FFFFFFF)),
                                 jnp.float32)
    out_ref[...] = ((1.0 - a) * new_ref[...]
                    + (a - a * a) * aligned_ref[...]
                    + (a * a) * s)


def _norm_body(x_ref, y_ref):
    x = x_ref[...]
    nrm = jnp.sqrt(jnp.sum(x * x, axis=1, keepdims=True))
    y_ref[...] = x / jnp.maximum(nrm, _EPS)


def _argmax_body(old_ref, newn_ref, idx_ref):
    sim = lax.dot_general(old_ref[...], newn_ref[...],
                          (((1,), (1,)), ((), ())),
                          preferred_element_type=jnp.float32)
    m = jnp.max(sim, axis=1, keepdims=True)
    col = lax.broadcasted_iota(jnp.int32, sim.shape, 1)
    cand = jnp.where(sim == m, col, sim.shape[1])
    idx_ref[...] = jnp.min(cand, axis=1, keepdims=True)


def _normalize(x):
    n, d = x.shape
    blk = min(n, 512)
    return pl.pallas_call(
        _norm_body,
        grid=(n // blk,),
        in_specs=[pl.BlockSpec((blk, d), lambda i: (i, 0))],
        out_specs=pl.BlockSpec((blk, d), lambda i: (i, 0)),
        out_shape=jax.ShapeDtypeStruct((n, d), jnp.float32),
    )(x)


def _top1(hidden_old, new_n):
    n, d = hidden_old.shape
    m = new_n.shape[0]
    blk = 256
    idx = pl.pallas_call(
        _argmax_body,
        grid=(n // blk,),
        in_specs=[pl.BlockSpec((blk, d), lambda i: (i, 0)),
                  pl.BlockSpec((m, d), lambda i: (0, 0))],
        out_specs=pl.BlockSpec((blk, 1), lambda i: (i, 0)),
        out_shape=jax.ShapeDtypeStruct((n, 1), jnp.int32),
    )(hidden_old, new_n)
    return idx.reshape(n)


def _colspec(n, blk):
    return pl.BlockSpec((n, blk), lambda i: (0, i))


def _rowspec(n, blk):
    return pl.BlockSpec((blk, n), lambda i: (i, 0))


def _ot_sort_all_body(old_ref, aligned_ref, new_ref, out_ref):
    a = _ALPHA
    n = old_ref.shape[0]
    rb = n.bit_length() - 1
    pb = 32 - rb
    rows = lax.broadcasted_iota(jnp.int32, (n, 1), 0)
    k1 = _sort_keys(_sortable(old_ref[...]))
    k2 = _sort_keys(
        (_sortable(aligned_ref[...]) & jnp.int32(~((1 << rb) - 1))) | rows)
    idx_tgt = k2 & jnp.int32((1 << rb) - 1)
    pay = (k1 >> rb) & jnp.int32((1 << pb) - 1)
    k3 = _sort_keys(((idx_tgt - jnp.int32(n // 2)) << pb) | pay)
    sb = (k3 & jnp.int32((1 << pb) - 1)) << rb
    s = lax.bitcast_convert_type(sb ^ ((sb >> 31) & jnp.int32(0x7FFFFFFF)),
                                 jnp.float32)
    out_ref[...] = ((1.0 - a) * new_ref[...]
                    + (a - a * a) * aligned_ref[...]
                    + (a * a) * s)


def _ot_update(hidden_old, aligned, hidden_new, blk=512):
    n, d = aligned.shape
    return pl.pallas_call(
        _ot_sort_all_body,
        grid=(d // blk,),
        in_specs=[_colspec(n, blk)] * 3,
        out_specs=_colspec(n, blk),
        out_shape=jax.ShapeDtypeStruct((n, d), jnp.float32),
    )(hidden_old, aligned, hidden_new)


def _gather_rows(table, idx):
    n, d = table.shape
    b = idx.shape[0]
    bpw = b // _NW
    mesh = plsc.VectorSubcoreMesh(core_axis_name="c", subcore_axis_name="s")

    @functools.partial(
        pl.kernel, mesh=mesh,
        out_type=jax.ShapeDtypeStruct((b, d), jnp.float32),
        scratch_types=[
            pltpu.VMEM((bpw,), jnp.int32),
            pltpu.VMEM((bpw, d), jnp.float32),
            pltpu.SemaphoreType.DMA,
        ],
    )
    def k(table_hbm, idx_hbm, out_hbm, idx_v, rows_v, sem):
        wid = lax.axis_index("s") * _NC + lax.axis_index("c")
        base = wid * bpw
        pltpu.sync_copy(idx_hbm.at[pl.ds(base, bpw)], idx_v)
        pltpu.async_copy(table_hbm.at[idx_v], rows_v, sem).wait()
        pltpu.sync_copy(rows_v, out_hbm.at[pl.ds(base, bpw)])

    return k(table, idx)


def kernel(hidden_old, hidden_new):
    new_n = _normalize(hidden_new)
    idx = _top1(hidden_old, new_n)
    aligned = _gather_rows(hidden_new, idx)
    return _ot_update(hidden_old, aligned, hidden_new)


# speedup vs baseline: 2.0373x; 1.0001x over previous
import functools

import jax
import jax.numpy as jnp
from jax import lax
from jax.experimental import pallas as pl
from jax.experimental.pallas import tpu as pltpu
from jax.experimental.pallas import tpu_sc as plsc

_ALPHA = 0.05
_EPS = 1e-8

_NC = 2
_NS = 16
_NW = _NC * _NS


def _tile_stage(v, lj, kb):
    j = 1 << lj
    r = lax.broadcasted_iota(jnp.int32, (1, 8, 1), 1)
    bj = (r >> lj) & 1
    tm = bj == kb
    p = jnp.where(bj == 1, pltpu.roll(v, j, 1), pltpu.roll(v, 8 - j, 1))
    return jnp.where((v < p) == tm, v, p)


def _sortable(f):
    b = lax.bitcast_convert_type(f, jnp.int32)
    return b ^ ((b >> 31) & jnp.int32(0x7FFFFFFF))


def _merge_rec(v, asc, ell):
    j = ell >> 1
    a, b = v[:, :j], v[:, j:]
    mn = jnp.minimum(a, b)
    mx = jnp.maximum(a, b)
    na = jnp.where(asc, mn, mx)
    nb = jnp.where(asc, mx, mn)
    if j == 8:
        return [na, nb]
    return _merge_rec(na, asc, j) + _merge_rec(nb, asc, j)


def _sort_keys(x):
    n, c = x.shape
    ln = n.bit_length() - 1
    nb8 = n // 8
    r = lax.broadcasted_iota(jnp.int32, (1, 8, 1), 1)
    t = lax.broadcasted_iota(jnp.int32, (nb8, 1, 1), 0)
    v = x.reshape(nb8, 8, c)
    for lk in (1, 2, 3):
        kb = ((r >> lk) & 1) if lk < 3 else (t & 1)
        for lj in range(lk - 1, -1, -1):
            v = _tile_stage(v, lj, kb)
    x = v.reshape(n, c)
    for lk in range(4, ln + 1):
        nb = n >> lk
        z = x.reshape(nb, 1 << lk, c)
        bm = lax.broadcasted_iota(jnp.int32, (nb, 1, 1), 0)
        kb = bm & 1
        pieces = _merge_rec(z, kb == 0, 1 << lk)
        done = []
        for p in pieces:
            for lj in (2, 1, 0):
                p = _tile_stage(p, lj, kb)
            done.append(p)
        x = jnp.stack(done, axis=1).reshape(n, c)
    return x


def _norm_body(x_ref, y_ref):
    x = x_ref[...]
    nrm = jnp.sqrt(jnp.sum(x * x, axis=1, keepdims=True))
    y_ref[...] = x / jnp.maximum(nrm, _EPS)


def _argmax_body(old_ref, newn_ref, idx_ref):
    sim = lax.dot_general(old_ref[...], newn_ref[...],
                          (((1,), (1,)), ((), ())),
                          preferred_element_type=jnp.float32)
    m = jnp.max(sim, axis=1, keepdims=True)
    col = lax.broadcasted_iota(jnp.int32, sim.shape, 1)
    cand = jnp.where(sim == m, col, sim.shape[1])
    idx_ref[...] = jnp.min(cand, axis=1, keepdims=True)


def _normalize(x):
    n, d = x.shape
    blk = min(n, 512)
    return pl.pallas_call(
        _norm_body,
        grid=(n // blk,),
        in_specs=[pl.BlockSpec((blk, d), lambda i: (i, 0))],
        out_specs=pl.BlockSpec((blk, d), lambda i: (i, 0)),
        out_shape=jax.ShapeDtypeStruct((n, d), jnp.float32),
    )(x)


def _top1(hidden_old, new_n):
    n, d = hidden_old.shape
    m = new_n.shape[0]
    blk = 256
    idx = pl.pallas_call(
        _argmax_body,
        grid=(n // blk,),
        in_specs=[pl.BlockSpec((blk, d), lambda i: (i, 0)),
                  pl.BlockSpec((m, d), lambda i: (0, 0))],
        out_specs=pl.BlockSpec((blk, 1), lambda i: (i, 0)),
        out_shape=jax.ShapeDtypeStruct((n, 1), jnp.int32),
    )(hidden_old, new_n)
    return idx.reshape(n)


def _colspec(n, blk):
    return pl.BlockSpec((n, blk), lambda i: (0, i))


def _ot_sort_all_body(old_ref, aligned_ref, new_ref, out_ref):
    a = _ALPHA
    n = old_ref.shape[0]
    rb = n.bit_length() - 1
    pb = 32 - rb
    rows = lax.broadcasted_iota(jnp.int32, (n, 1), 0)
    k1 = _sort_keys(_sortable(old_ref[...]))
    k2 = _sort_keys(
        (_sortable(aligned_ref[...]) & jnp.int32(~((1 << rb) - 1))) | rows)
    idx_tgt = k2 & jnp.int32((1 << rb) - 1)
    pay = (k1 >> rb) & jnp.int32((1 << pb) - 1)
    k3 = _sort_keys(((idx_tgt - jnp.int32(n // 2)) << pb) | pay)
    sb = (k3 & jnp.int32((1 << pb) - 1)) << rb
    s = lax.bitcast_convert_type(sb ^ ((sb >> 31) & jnp.int32(0x7FFFFFFF)),
                                 jnp.float32)
    out_ref[...] = ((1.0 - a) * new_ref[...]
                    + (a - a * a) * aligned_ref[...]
                    + (a * a) * s)


def _ot_update(hidden_old, aligned, hidden_new, blk=512):
    n, d = aligned.shape
    return pl.pallas_call(
        _ot_sort_all_body,
        grid=(d // blk,),
        in_specs=[_colspec(n, blk)] * 3,
        out_specs=_colspec(n, blk),
        out_shape=jax.ShapeDtypeStruct((n, d), jnp.float32),
    )(hidden_old, aligned, hidden_new)


def _gather_rows(table, idx):
    n, d = table.shape
    b = idx.shape[0]
    bpw = b // _NW
    mesh = plsc.VectorSubcoreMesh(core_axis_name="c", subcore_axis_name="s")

    @functools.partial(
        pl.kernel, mesh=mesh,
        out_type=jax.ShapeDtypeStruct((b, d), jnp.float32),
        scratch_types=[
            pltpu.VMEM((bpw,), jnp.int32),
            pltpu.VMEM((bpw, d), jnp.float32),
            pltpu.SemaphoreType.DMA,
        ],
    )
    def k(table_hbm, idx_hbm, out_hbm, idx_v, rows_v, sem):
        wid = lax.axis_index("s") * _NC + lax.axis_index("c")
        base = wid * bpw
        pltpu.sync_copy(idx_hbm.at[pl.ds(base, bpw)], idx_v)
        pltpu.async_copy(table_hbm.at[idx_v], rows_v, sem).wait()
        pltpu.sync_copy(rows_v, out_hbm.at[pl.ds(base, bpw)])

    return k(table, idx)


def kernel(hidden_old, hidden_new):
    new_n = _normalize(hidden_new)
    idx = _top1(hidden_old, new_n)
    aligned = _gather_rows(hidden_new, idx)
    return _ot_update(hidden_old, aligned, hidden_new)
